# Initial kernel scaffold; baseline (speedup 1.0000x reference)
import jax
import jax.numpy as jnp
from jax.experimental import pallas as pl


def kernel(x, vertices, conv_w, conv_b, r_w1, r_b1, r_w2, r_b2, r_w3, r_b3, lin_w, lin_b, bn_gamma, bn_beta):
    raise NotImplementedError("write your pallas kernel here")


# speedup vs baseline: 5.8236x; 5.8236x over previous
import functools

import jax
import jax.numpy as jnp
from jax.experimental import pallas as pl
from jax.experimental.pallas import tpu as pltpu

VIEWS = 32
K = 5
IMG = 3 * 34 * 18
FEAT = 512

GB = 8
ROWS = GB * VIEWS


def _prep_body(v_ref, cw_ref, cb_ref, w1_ref, b1_ref, w2_ref, b2_ref,
               w3_ref, b3_ref, linT_ref, lb_ref, a_out, bias_out, wcl_out):
    V = v_ref[...]
    G = jax.lax.dot_general(V, V, (((1,), (1,)), ((), ())),
                            preferred_element_type=jnp.float32)
    ii = jax.lax.broadcasted_iota(jnp.int32, (VIEWS, VIEWS), 0)
    jj = jax.lax.broadcasted_iota(jnp.int32, (VIEWS, VIEWS), 1)
    diagmask = ii == jj
    xx_col = jnp.sum(jnp.where(diagmask, G, 0.0), axis=1, keepdims=True)
    xx_row = jnp.sum(jnp.where(diagmask, G, 0.0), axis=0, keepdims=True)
    nd = 2.0 * G - xx_col - xx_row

    b1 = b1_ref[...]
    b2 = b2_ref[...]
    b3 = b3_ref[...]
    A = jnp.zeros((VIEWS, VIEWS), jnp.float32)
    v0 = None
    for k in range(K):
        m = jnp.max(nd, axis=1, keepdims=True)
        cand = jnp.where(nd >= m, jj, jnp.int32(2 ** 30))
        idxk = jnp.min(cand, axis=1, keepdims=True)
        onehot = jnp.where(jj == idxk, 1.0, 0.0)
        vk = jnp.dot(onehot, V, preferred_element_type=jnp.float32)
        if k == 0:
            v0 = vk
        diff = v0 - vk
        nrm = jnp.sqrt(jnp.sum(diff * diff, axis=1, keepdims=True))
        h = (jnp.dot(v0, w1_ref[0:3, :], preferred_element_type=jnp.float32)
             + jnp.dot(vk, w1_ref[3:6, :], preferred_element_type=jnp.float32)
             + jnp.dot(diff, w1_ref[6:9, :], preferred_element_type=jnp.float32)
             + nrm * w1_ref[9:10, :] + b1)
        h = jnp.maximum(h, 0.0)
        h = jnp.maximum(jnp.dot(h, w2_ref[...],
                                preferred_element_type=jnp.float32) + b2, 0.0)
        wk = jnp.dot(h, w3_ref[...], preferred_element_type=jnp.float32) + b3
        A = A + wk * onehot
        nd = jnp.where(jj == idxk, jnp.float32(-1e30), nd)
    a_out[...] = A

    s = jnp.sum(A, axis=1, keepdims=True)
    t = jnp.sum(linT_ref[...], axis=0, keepdims=True)
    bias_out[...] = cb_ref[...] * s * t + lb_ref[...]

    L1 = jax.lax.broadcasted_iota(jnp.int32, (IMG, 1), 0)
    mcol = jax.lax.broadcasted_iota(jnp.int32, (1, FEAT), 1)
    r_ = mcol // 16
    c_ = mcol % 16
    Wc = jnp.zeros((IMG, FEAT), jnp.float32)
    for ci in range(3):
        for dr in range(3):
            for dc in range(3):
                tgt = ci * 612 + (r_ + dr) * 18 + (c_ + dc)
                kk = ci * 9 + dr * 3 + dc
                Wc = Wc + jnp.where(L1 == tgt, cw_ref[0:1, kk:kk + 1], 0.0)
    wcl_out[...] = jnp.dot(Wc, linT_ref[...], preferred_element_type=jnp.float32)


def _main_body(x_ref, wcl_ref, a_ref, bias_ref, f_ref, st_ref):
    P = jnp.dot(x_ref[...], wcl_ref[...],
                preferred_element_type=jnp.float32)
    Av = a_ref[...]
    Bv = bias_ref[...]
    ssum = jnp.zeros((1, FEAT), jnp.float32)
    ssq = jnp.zeros((1, FEAT), jnp.float32)
    for g in range(GB):
        Z = jnp.dot(Av, P[g * VIEWS:(g + 1) * VIEWS, :],
                    preferred_element_type=jnp.float32) + Bv
        Fg = jnp.maximum(Z, 0.0)
        f_ref[g * VIEWS:(g + 1) * VIEWS, :] = Fg
        ssum = ssum + jnp.sum(Fg, axis=0, keepdims=True)
        ssq = ssq + jnp.sum(Fg * Fg, axis=0, keepdims=True)
    st_ref[0, 0:1, :] = ssum
    st_ref[0, 1:2, :] = ssq


def _apply_body(n_total, f_ref, st_ref, g_ref, b_ref, o_ref):
    st = st_ref[...]
    inv_n = jnp.float32(1.0 / n_total)
    mean = jnp.sum(st[:, 0, :], axis=0, keepdims=True) * inv_n
    msq = jnp.sum(st[:, 1, :], axis=0, keepdims=True) * inv_n
    var = msq - mean * mean
    scale = g_ref[...] * jax.lax.rsqrt(var + 1e-5)
    shift = b_ref[...] - mean * scale
    o_ref[...] = f_ref[...] * scale + shift


def kernel(x, vertices, conv_w, conv_b, r_w1, r_b1, r_w2, r_b2, r_w3, r_b3,
           lin_w, lin_b, bn_gamma, bn_beta):
    n = x.shape[0]
    xf = x.reshape(n, IMG)
    lin_wT = lin_w.T

    a_mat, bias, wcl = pl.pallas_call(
        _prep_body,
        out_shape=[
            jax.ShapeDtypeStruct((VIEWS, VIEWS), jnp.float32),
            jax.ShapeDtypeStruct((VIEWS, FEAT), jnp.float32),
            jax.ShapeDtypeStruct((IMG, FEAT), jnp.float32),
        ],
    )(vertices, conv_w.reshape(1, 27), conv_b.reshape(1, 1),
      r_w1.T, r_b1.reshape(1, 10), r_w2.T, r_b2.reshape(1, 10),
      r_w3.T, r_b3.reshape(1, 1), lin_wT, lin_b.reshape(1, FEAT))

    nsteps = n // ROWS
    f_pre, stats = pl.pallas_call(
        _main_body,
        grid=(nsteps,),
        in_specs=[
            pl.BlockSpec((ROWS, IMG), lambda i: (i, 0)),
            pl.BlockSpec((IMG, FEAT), lambda i: (0, 0)),
            pl.BlockSpec((VIEWS, VIEWS), lambda i: (0, 0)),
            pl.BlockSpec((VIEWS, FEAT), lambda i: (0, 0)),
        ],
        out_specs=[
            pl.BlockSpec((ROWS, FEAT), lambda i: (i, 0)),
            pl.BlockSpec((1, 8, FEAT), lambda i: (i, 0, 0)),
        ],
        out_shape=[
            jax.ShapeDtypeStruct((n, FEAT), jnp.float32),
            jax.ShapeDtypeStruct((nsteps, 8, FEAT), jnp.float32),
        ],
        compiler_params=pltpu.CompilerParams(
            dimension_semantics=("parallel",)),
    )(xf, wcl, a_mat, bias)

    arows = 2048
    out = pl.pallas_call(
        functools.partial(_apply_body, n),
        grid=(n // arows,),
        in_specs=[
            pl.BlockSpec((arows, FEAT), lambda i: (i, 0)),
            pl.BlockSpec((nsteps, 8, FEAT), lambda i: (0, 0, 0)),
            pl.BlockSpec((1, FEAT), lambda i: (0, 0)),
            pl.BlockSpec((1, FEAT), lambda i: (0, 0)),
        ],
        out_specs=pl.BlockSpec((arows, FEAT), lambda i: (i, 0)),
        out_shape=jax.ShapeDtypeStruct((n, FEAT), jnp.float32),
        compiler_params=pltpu.CompilerParams(
            dimension_semantics=("parallel",)),
    )(f_pre, stats, bn_gamma.reshape(1, FEAT), bn_beta.reshape(1, FEAT))

    return out.reshape(n, 1, FEAT)
